# baseline (speedup 1.0000x reference)
import jax
import jax.numpy as jnp
from jax import lax
from jax.experimental import pallas as pl
from jax.experimental.pallas import tpu as pltpu
from jax.experimental.pallas import tpu_sc as plsc

N = 10000
D = 128
NP = 10112
RPS = NP // 16
CHUNK = 128
NBUF = 4
NGB = 2


def _cdiv(a, b):
    return (a + b - 1) // b


def _make_phase(half, with_counts, seg_core):
    cpt = 2 * half
    mesh = plsc.VectorSubcoreMesh(core_axis_name="c", subcore_axis_name="s")

    def body(*refs):
        if with_counts:
            (table, src3d, dst2d, dstc2d, z128, p_out, c_out,
             sidx, didx, rows, acc, gs0, gs1, is0, is1, is2, is3) = refs
        else:
            (table, src3d, dst2d, z128, p_out,
             sidx, didx, rows, acc, gs0, gs1, is0, is1, is2, is3) = refs
        gsem = (gs0, gs1)
        isem = (is0, is1, is2, is3)
        c = lax.axis_index("c")
        s = lax.axis_index("s")
        r0 = s * RPS

        def idx_start(chunk, bi):
            pltpu.async_copy(src3d.at[chunk], sidx.at[pl.ds(bi, 1)], isem[bi])

        def idx_wait(bi):
            pltpu.make_async_copy(src3d.at[0],
                                  sidx.at[pl.ds(bi, 1)], isem[bi]).wait()

        def gather_start(bi, bg):
            pltpu.async_copy(table.at[sidx.at[bi]], rows.at[bg], gsem[bg])

        def gather_wait(bg):
            pltpu.make_async_copy(table.at[sidx.at[0]],
                                  rows.at[bg], gsem[bg]).wait()

        def seg_half(h):
            base = s * cpt + h * half
            pltpu.sync_copy(dst2d.at[pl.ds(base, half)],
                            didx.at[pl.ds(0, half)])
            for j in range(NBUF):
                idx_start(base + j, j)
            for j in range(NGB):
                idx_wait(j)
                gather_start(j, j)

            def group(g, _):
                for b in range(NBUF):
                    lc = g * NBUF + b
                    bg = b % NGB
                    gather_wait(bg)
                    pltpu.sync_copy(rows.at[bg], acc.at[didx.at[lc]],
                                    add=True)
                    ni = lc + NBUF

                    @pl.when(ni < half)
                    def _():
                        idx_start(base + ni, b)
                    ng = lc + NGB

                    @pl.when(ng < half)
                    def _():
                        idx_wait((b + NGB) % NBUF)
                        gather_start((b + NGB) % NBUF, bg)
                return 0
            lax.fori_loop(0, half // NBUF, group, 0)

        def cnt_half(h):
            pltpu.sync_copy(dstc2d.at[pl.ds(s * cpt + h * half, half)],
                            didx.at[pl.ds(0, half)])

            def stepc(j, _):
                pltpu.sync_copy(rows.at[0], acc.at[didx.at[j]], add=True)
                return 0
            lax.fori_loop(0, half, stepc, 0)

        if with_counts:
            pltpu.sync_copy(z128.at[pl.ds(r0, RPS)], acc.at[pl.ds(r0, RPS)])
        else:
            @pl.when(c == seg_core)
            def _():
                pltpu.sync_copy(z128.at[pl.ds(r0, RPS)],
                                acc.at[pl.ds(r0, RPS)])

        if with_counts:
            @pl.when(c != seg_core)
            def _():
                def fill(r, _):
                    for k in range(D // 16):
                        rows[0, r, pl.ds(k * 16, 16)] = jnp.ones(
                            (16,), jnp.float32)
                    return 0
                lax.fori_loop(0, CHUNK, fill, 0)
        plsc.subcore_barrier()

        @pl.when(c == seg_core)
        def _():
            seg_half(0)
            seg_half(1)

        if with_counts:
            @pl.when(c != seg_core)
            def _():
                cnt_half(0)
                cnt_half(1)
        plsc.subcore_barrier()

        @pl.when(c == seg_core)
        def _():
            pltpu.sync_copy(acc.at[pl.ds(r0, RPS)], p_out.at[pl.ds(r0, RPS)])

        if with_counts:
            @pl.when(c != seg_core)
            def _():
                pltpu.sync_copy(acc.at[pl.ds(r0, RPS)],
                                c_out.at[pl.ds(r0, RPS)])

    out_type = [jax.ShapeDtypeStruct((NP, D), jnp.float32)]
    if with_counts:
        out_type.append(jax.ShapeDtypeStruct((NP, D), jnp.float32))
    return pl.kernel(
        body,
        out_type=out_type,
        mesh=mesh,
        scratch_types=[
            pltpu.VMEM((NBUF, CHUNK), jnp.int32),
            pltpu.VMEM((half, CHUNK), jnp.int32),
            pltpu.VMEM((NGB, CHUNK, D), jnp.float32),
            pltpu.VMEM_SHARED((NP, D), jnp.float32),
            pltpu.SemaphoreType.DMA,
            pltpu.SemaphoreType.DMA,
            pltpu.SemaphoreType.DMA,
            pltpu.SemaphoreType.DMA,
            pltpu.SemaphoreType.DMA,
            pltpu.SemaphoreType.DMA,
        ],
    )


def _pad_edges(ei, total_chunks):
    total = total_chunks * CHUNK
    pad = total - ei.shape[1]
    src = jnp.concatenate([ei[0], jnp.zeros((pad,), jnp.int32)])
    dst = jnp.concatenate([ei[1], jnp.full((pad,), N, jnp.int32)])
    return src.reshape(-1, 1, CHUNK), dst.reshape(-1, CHUNK)


_ROWS = 1000


def _mm_t(a, w):
    return lax.dot_general(a, w, (((1,), (1,)), ((), ())),
                           preferred_element_type=jnp.float32)


def _sage_body(p_ref, c_ref, x_ref, wl_ref, bl_ref, wr_ref, o_ref):
    cnt = c_ref[:, 0:1]
    m = p_ref[...] / jnp.maximum(cnt, 1.0)
    o = _mm_t(m, wl_ref[...]) + bl_ref[...] + _mm_t(x_ref[...], wr_ref[...])
    nrm = jnp.sqrt(jnp.sum(o * o, axis=-1, keepdims=True))
    o = o / jnp.maximum(nrm, 1e-12)
    o_ref[...] = jnp.maximum(o, 0.0)


def _head_body(p_ref, c_ref, wlin_ref, blin_ref, o_ref):
    cnt = c_ref[:, 0:1]
    m = p_ref[...] / jnp.maximum(cnt, 1.0)
    o_ref[...] = _mm_t(m, wlin_ref[...]) + blin_ref[...]


def _sage_tc(p, cnt, x, Wl, bl, Wr):
    grid = (N // _ROWS,)
    return pl.pallas_call(
        _sage_body,
        grid=grid,
        in_specs=[
            pl.BlockSpec((_ROWS, D), lambda i: (i, 0)),
            pl.BlockSpec((_ROWS, D), lambda i: (i, 0)),
            pl.BlockSpec((_ROWS, D), lambda i: (i, 0)),
            pl.BlockSpec((D, D), lambda i: (0, 0)),
            pl.BlockSpec((1, D), lambda i: (0, 0)),
            pl.BlockSpec((D, D), lambda i: (0, 0)),
        ],
        out_specs=pl.BlockSpec((_ROWS, D), lambda i: (i, 0)),
        out_shape=jax.ShapeDtypeStruct((N, D), jnp.float32),
    )(p, cnt, x, Wl, bl.reshape(1, D), Wr)


def _head_tc(p, cnt, Wlin, blin):
    grid = (N // _ROWS,)
    return pl.pallas_call(
        _head_body,
        grid=grid,
        in_specs=[
            pl.BlockSpec((_ROWS, D), lambda i: (i, 0)),
            pl.BlockSpec((_ROWS, D), lambda i: (i, 0)),
            pl.BlockSpec((D, D), lambda i: (0, 0)),
            pl.BlockSpec((1, D), lambda i: (0, 0)),
        ],
        out_specs=pl.BlockSpec((_ROWS, D), lambda i: (i, 0)),
        out_shape=jax.ShapeDtypeStruct((N, D), jnp.float32),
    )(p, cnt, Wlin, blin.reshape(1, D))


def kernel(x, edge_index, k_hop_edge_index, Wl1, bl1, Wr1, Wl2, bl2, Wr2,
           Wlin, blin):
    nch_e = _cdiv(edge_index.shape[1], CHUNK)
    nch_k = _cdiv(k_hop_edge_index.shape[1], CHUNK)
    half = max(_cdiv(nch_e, 16 * 2 * 8) * 8, _cdiv(nch_k, 16 * 2 * 8) * 8)
    total_chunks = 16 * 2 * half
    srcE, dstE = _pad_edges(edge_index, total_chunks)
    srcK, dstK = _pad_edges(k_hop_edge_index, total_chunks)
    z128 = jnp.zeros((NP, D), jnp.float32)

    phase_a = _make_phase(half, True, 0)
    phase_b = _make_phase(half, True, 1)
    phase_cc = _make_phase(half, False, 0)

    p1, ce = phase_a(x, srcE, dstE, dstE, z128)
    h1 = _sage_tc(p1, ce, x, Wl1, bl1, Wr1)
    p2, ck = phase_b(h1, srcE, dstE, dstK, z128)
    h2 = _sage_tc(p2, ce, h1, Wl2, bl2, Wr2)
    (p3,) = phase_cc(h2, srcK, dstK, z128)
    return _head_tc(p3, ck, Wlin, blin)


# speedup vs baseline: 1.0674x; 1.0674x over previous
import jax
import jax.numpy as jnp
from jax import lax
from jax.experimental import pallas as pl
from jax.experimental.pallas import tpu as pltpu
from jax.experimental.pallas import tpu_sc as plsc

N = 10000
D = 128
NP = 10112
RPS = NP // 16
CHUNK = 128
NTILES = 32
NBUF = 4
NGB = 2


def _cdiv(a, b):
    return (a + b - 1) // b


def _make_seg(cpt):
    assert cpt % NBUF == 0 and cpt % 8 == 0
    mesh = plsc.VectorSubcoreMesh(core_axis_name="c", subcore_axis_name="s")

    def body(table, src3d, dst2d, z128, out,
             sidx, didx, rows, acc, gs0, gs1, is0, is1, is2, is3):
        gsem = (gs0, gs1)
        isem = (is0, is1, is2, is3)
        c = lax.axis_index("c")
        s = lax.axis_index("s")
        wid = s * 2 + c
        r0 = s * RPS
        base = wid * cpt

        def idx_start(chunk, bi):
            pltpu.async_copy(src3d.at[base + chunk],
                             sidx.at[pl.ds(bi, 1)], isem[bi])

        def idx_wait(bi):
            pltpu.make_async_copy(src3d.at[0],
                                  sidx.at[pl.ds(bi, 1)], isem[bi]).wait()

        def gather_start(bi, bg):
            pltpu.async_copy(table.at[sidx.at[bi]], rows.at[bg], gsem[bg])

        def gather_wait(bg):
            pltpu.make_async_copy(table.at[sidx.at[0]],
                                  rows.at[bg], gsem[bg]).wait()

        for j in range(NBUF):
            idx_start(j, j)
        for j in range(NGB):
            idx_wait(j)
            gather_start(j, j)
        pltpu.sync_copy(dst2d.at[pl.ds(base, cpt)], didx)
        pltpu.sync_copy(z128.at[pl.ds(r0, RPS)], acc.at[pl.ds(r0, RPS)])
        plsc.subcore_barrier()

        def group(g, _):
            for b in range(NBUF):
                chunk = g * NBUF + b
                bg = b % NGB
                gather_wait(bg)
                pltpu.sync_copy(rows.at[bg], acc.at[didx.at[chunk]],
                                add=True)
                ni = chunk + NBUF

                @pl.when(ni < cpt)
                def _():
                    idx_start(ni, b)
                ng = chunk + NGB

                @pl.when(ng < cpt)
                def _():
                    idx_wait((b + NGB) % NBUF)
                    gather_start((b + NGB) % NBUF, bg)
            return 0
        lax.fori_loop(0, cpt // NBUF, group, 0)

        plsc.subcore_barrier()
        pltpu.sync_copy(acc.at[pl.ds(r0, RPS)], out.at[c, pl.ds(r0, RPS)])

    return pl.kernel(
        body,
        out_type=[jax.ShapeDtypeStruct((2, NP, D), jnp.float32)],
        mesh=mesh,
        scratch_types=[
            pltpu.VMEM((NBUF, CHUNK), jnp.int32),
            pltpu.VMEM((cpt, CHUNK), jnp.int32),
            pltpu.VMEM((NGB, CHUNK, D), jnp.float32),
            pltpu.VMEM_SHARED((NP, D), jnp.float32),
            pltpu.SemaphoreType.DMA,
            pltpu.SemaphoreType.DMA,
            pltpu.SemaphoreType.DMA,
            pltpu.SemaphoreType.DMA,
            pltpu.SemaphoreType.DMA,
            pltpu.SemaphoreType.DMA,
        ],
    )


def _make_counts(cpt):
    mesh = plsc.VectorSubcoreMesh(core_axis_name="c", subcore_axis_name="s")

    def body(dstE, dstK, z128, o128, ce_out, ck_out, didx, ones, acc):
        c = lax.axis_index("c")
        s = lax.axis_index("s")
        wid = s * 2 + c
        r0 = s * RPS
        pltpu.sync_copy(o128.at[pl.ds(0, CHUNK)], ones)

        for dst, out in ((dstE, ce_out), (dstK, ck_out)):
            pltpu.sync_copy(dst.at[pl.ds(wid * cpt, cpt)], didx)
            pltpu.sync_copy(z128.at[pl.ds(r0, RPS)], acc.at[pl.ds(r0, RPS)])
            plsc.subcore_barrier()

            def step(j, _):
                pltpu.sync_copy(ones, acc.at[didx.at[j]], add=True)
                return 0
            lax.fori_loop(0, cpt, step, 0)

            plsc.subcore_barrier()
            pltpu.sync_copy(acc.at[pl.ds(r0, RPS)], out.at[c, pl.ds(r0, RPS)])

    return pl.kernel(
        body,
        out_type=[jax.ShapeDtypeStruct((2, NP, D), jnp.float32),
                  jax.ShapeDtypeStruct((2, NP, D), jnp.float32)],
        mesh=mesh,
        scratch_types=[
            pltpu.VMEM((cpt, CHUNK), jnp.int32),
            pltpu.VMEM((CHUNK, D), jnp.float32),
            pltpu.VMEM_SHARED((NP, D), jnp.float32),
        ],
    )


def _pad_edges(ei, total_chunks):
    total = total_chunks * CHUNK
    pad = total - ei.shape[1]
    src = jnp.concatenate([ei[0], jnp.zeros((pad,), jnp.int32)])
    dst = jnp.concatenate([ei[1], jnp.full((pad,), N, jnp.int32)])
    return src.reshape(-1, 1, CHUNK), dst.reshape(-1, CHUNK)


_ROWS = 1000


def _mm_t(a, w):
    return lax.dot_general(a, w, (((1,), (1,)), ((), ())),
                           preferred_element_type=jnp.float32)


def _sage_body(p_ref, c_ref, x_ref, wl_ref, bl_ref, wr_ref, o_ref):
    ssum = p_ref[0] + p_ref[1]
    cnt = c_ref[0, :, 0:1] + c_ref[1, :, 0:1]
    m = ssum / jnp.maximum(cnt, 1.0)
    o = _mm_t(m, wl_ref[...]) + bl_ref[...] + _mm_t(x_ref[...], wr_ref[...])
    nrm = jnp.sqrt(jnp.sum(o * o, axis=-1, keepdims=True))
    o = o / jnp.maximum(nrm, 1e-12)
    o_ref[...] = jnp.maximum(o, 0.0)


def _head_body(p_ref, c_ref, wlin_ref, blin_ref, o_ref):
    ssum = p_ref[0] + p_ref[1]
    cnt = c_ref[0, :, 0:1] + c_ref[1, :, 0:1]
    m = ssum / jnp.maximum(cnt, 1.0)
    o_ref[...] = _mm_t(m, wlin_ref[...]) + blin_ref[...]


def _sage_tc(p, cnt, x, Wl, bl, Wr):
    grid = (N // _ROWS,)
    return pl.pallas_call(
        _sage_body,
        grid=grid,
        in_specs=[
            pl.BlockSpec((2, _ROWS, D), lambda i: (0, i, 0)),
            pl.BlockSpec((2, _ROWS, D), lambda i: (0, i, 0)),
            pl.BlockSpec((_ROWS, D), lambda i: (i, 0)),
            pl.BlockSpec((D, D), lambda i: (0, 0)),
            pl.BlockSpec((1, D), lambda i: (0, 0)),
            pl.BlockSpec((D, D), lambda i: (0, 0)),
        ],
        out_specs=pl.BlockSpec((_ROWS, D), lambda i: (i, 0)),
        out_shape=jax.ShapeDtypeStruct((N, D), jnp.float32),
    )(p, cnt, x, Wl, bl.reshape(1, D), Wr)


def _head_tc(p, cnt, Wlin, blin):
    grid = (N // _ROWS,)
    return pl.pallas_call(
        _head_body,
        grid=grid,
        in_specs=[
            pl.BlockSpec((2, _ROWS, D), lambda i: (0, i, 0)),
            pl.BlockSpec((2, _ROWS, D), lambda i: (0, i, 0)),
            pl.BlockSpec((D, D), lambda i: (0, 0)),
            pl.BlockSpec((1, D), lambda i: (0, 0)),
        ],
        out_specs=pl.BlockSpec((_ROWS, D), lambda i: (i, 0)),
        out_shape=jax.ShapeDtypeStruct((N, D), jnp.float32),
    )(p, cnt, Wlin, blin.reshape(1, D))


def kernel(x, edge_index, k_hop_edge_index, Wl1, bl1, Wr1, Wl2, bl2, Wr2,
           Wlin, blin):
    nch = max(_cdiv(edge_index.shape[1], CHUNK),
              _cdiv(k_hop_edge_index.shape[1], CHUNK))
    cpt = _cdiv(nch, NTILES * 8) * 8
    total_chunks = NTILES * cpt
    srcE, dstE = _pad_edges(edge_index, total_chunks)
    srcK, dstK = _pad_edges(k_hop_edge_index, total_chunks)
    z128 = jnp.zeros((NP, D), jnp.float32)
    o128 = jnp.ones((CHUNK, D), jnp.float32)

    seg = _make_seg(cpt)
    counts = _make_counts(cpt)

    ce, ck = counts(dstE, dstK, z128, o128)
    (p1,) = seg(x, srcE, dstE, z128)
    h1 = _sage_tc(p1, ce, x, Wl1, bl1, Wr1)
    (p2,) = seg(h1, srcE, dstE, z128)
    h2 = _sage_tc(p2, ce, h1, Wl2, bl2, Wr2)
    (p3,) = seg(h2, srcK, dstK, z128)
    return _head_tc(p3, ck, Wlin, blin)
